# baseline (speedup 1.0000x reference)
import functools

import jax
import jax.numpy as jnp
from jax import lax
from jax.experimental import pallas as pl
from jax.experimental.pallas import tpu as pltpu
from jax.experimental.pallas import tpu_sc as plsc

_NUM_HEADS = 16
_MAX_DISTANCE = 128


def _build_wt_sc(table_flat, q_len, n_heads):
    max_rel = _MAX_DISTANCE - 1
    info = plsc.get_sparse_core_info()
    nc, ns = info.num_cores, info.num_subcores
    nw = nc * ns
    w_cols = 2 * q_len
    halves = nw // n_heads
    cols_per_w = w_cols // halves

    mesh = plsc.VectorSubcoreMesh(core_axis_name="c", subcore_axis_name="s")

    @functools.partial(
        pl.kernel,
        mesh=mesh,
        out_type=jax.ShapeDtypeStruct((n_heads, w_cols), jnp.float32),
        scratch_types=[
            pltpu.VMEM((table_flat.shape[0],), jnp.float32),
            pltpu.VMEM((cols_per_w,), jnp.float32),
            pltpu.SemaphoreType.DMA,
        ],
        compiler_params=pltpu.CompilerParams(
            use_tc_tiling_on_sc=False, needs_layout_passes=False
        ),
    )
    def sc_kernel(table_hbm, wt_hbm, table_v, row_v, sem):
        wid = lax.axis_index("s") * nc + lax.axis_index("c")
        h = wid // halves
        half = wid % halves

        pltpu.sync_copy(table_hbm, table_v)

        lane = lax.iota(jnp.int32, 16)
        u_base = half * cols_per_w

        def build(k, carry):
            u = u_base + k * 16 + lane
            rel = (q_len - 1) - u
            idx = jnp.clip(rel, -max_rel, max_rel) + max_rel
            vals = plsc.load_gather(table_v, [idx * n_heads + h])
            row_v[pl.ds(k * 16, 16)] = vals
            return carry

        lax.fori_loop(0, cols_per_w // 16, build, 0)

        pltpu.sync_copy(row_v, wt_hbm.at[h, pl.ds(u_base, cols_per_w)])

    return sc_kernel(table_flat)


def kernel(x, relative_attention_bias_table):
    q_len = x.shape[1]
    n_table, n_heads = relative_attention_bias_table.shape

    wt = _build_wt_sc(relative_attention_bias_table.reshape(-1), q_len, n_heads)

    bq = 128

    def tc_body(wt_ref, out_ref):
        ib = pl.program_id(0)
        start = pl.multiple_of(q_len - bq * (ib + 1), bq)
        cat = wt_ref[:, pl.ds(start, q_len + bq)]
        for r in range(bq):
            shifted = pltpu.roll(cat, (q_len + bq) + r - (bq - 1), axis=1)
            out_ref[r] = shifted[:, :q_len]

    out_t = pl.pallas_call(
        tc_body,
        grid=(q_len // bq,),
        in_specs=[pl.BlockSpec((n_heads, 2 * q_len), lambda i: (0, 0))],
        out_specs=pl.BlockSpec((bq, n_heads, q_len), lambda i: (i, 0, 0)),
        out_shape=jax.ShapeDtypeStruct((q_len, n_heads, q_len), jnp.float32),
    )(wt)

    return jnp.transpose(out_t, (0, 2, 1))


# speedup vs baseline: 1.0207x; 1.0207x over previous
import functools

import jax
import jax.numpy as jnp
from jax import lax
from jax.experimental import pallas as pl
from jax.experimental.pallas import tpu as pltpu
from jax.experimental.pallas import tpu_sc as plsc

_NUM_HEADS = 16
_MAX_DISTANCE = 128


def _build_wt_sc(table_flat, n_table, q_len, n_heads):
    max_rel = _MAX_DISTANCE - 1
    info = plsc.get_sparse_core_info()
    nc, ns = info.num_cores, info.num_subcores
    nw = nc * ns
    w_cols = 2 * q_len
    halves = nw // n_heads
    cols_per_w = w_cols // halves

    mesh = plsc.VectorSubcoreMesh(core_axis_name="c", subcore_axis_name="s")

    @functools.partial(
        pl.kernel,
        mesh=mesh,
        out_type=jax.ShapeDtypeStruct((n_heads, w_cols), jnp.float32),
        scratch_types=[
            pltpu.VMEM((table_flat.shape[0],), jnp.float32),
            pltpu.VMEM((cols_per_w,), jnp.float32),
            pltpu.SemaphoreType.DMA,
        ],
        compiler_params=pltpu.CompilerParams(
            use_tc_tiling_on_sc=False, needs_layout_passes=False
        ),
    )
    def sc_kernel(table_hbm, wt_hbm, table_v, row_v, sem):
        wid = lax.axis_index("s") * nc + lax.axis_index("c")
        h = wid // halves
        half = wid % halves

        pltpu.sync_copy(table_hbm, table_v)

        lane = lax.iota(jnp.int32, 16)
        u_base = half * cols_per_w

        def build(k, carry):
            u = u_base + k * 16 + lane
            rel = (q_len - 1) - u
            idx = jnp.clip(rel, -max_rel, max_rel) + max_rel
            vals = plsc.load_gather(table_v, [h * n_table + idx])
            row_v[pl.ds(k * 16, 16)] = vals
            return carry

        lax.fori_loop(0, cols_per_w // 16, build, 0)

        pltpu.sync_copy(row_v, wt_hbm.at[h, pl.ds(u_base, cols_per_w)])

    return sc_kernel(table_flat)


def kernel(x, relative_attention_bias_table):
    q_len = x.shape[1]
    n_table, n_heads = relative_attention_bias_table.shape

    wt = _build_wt_sc(
        relative_attention_bias_table.T.reshape(-1), n_table, q_len, n_heads
    )

    bq = 128

    def tc_body(wt_ref, out_ref):
        ib = pl.program_id(0)
        start = pl.multiple_of(q_len - bq * (ib + 1), bq)
        cat = wt_ref[:, pl.ds(start, q_len + bq)]
        for r in range(bq):
            shifted = pltpu.roll(cat, (q_len + bq) + r - (bq - 1), axis=1)
            out_ref[r] = shifted[:, :q_len]

    out_t = pl.pallas_call(
        tc_body,
        grid=(q_len // bq,),
        in_specs=[pl.BlockSpec((n_heads, 2 * q_len), lambda i: (0, 0))],
        out_specs=pl.BlockSpec((bq, n_heads, q_len), lambda i: (i, 0, 0)),
        out_shape=jax.ShapeDtypeStruct((q_len, n_heads, q_len), jnp.float32),
    )(wt)

    return jnp.transpose(out_t, (0, 2, 1))
